# baseline (speedup 1.0000x reference)
import functools

import jax
import jax.numpy as jnp
from jax import lax
from jax.experimental import pallas as pl
from jax.experimental.pallas import tpu as pltpu
from jax.experimental.pallas import tpu_sc as plsc

NUM_UNITS = 64
SCALE = 8.0
NW = 32
CHUNK = 128
LANES = 16
NBUF = 2
PF = 1


def _sc_body(idx_hbm, table_hbm, out_hbm, idx_v, pair_v, rows_v, comp_v, gsem, ssem):
    nchunk = idx_v.shape[0]
    s_tiles = out_hbm.shape[1] // 8
    b_tiles = out_hbm.shape[0] // CHUNK
    wid = lax.axis_index("s") * 2 + lax.axis_index("c")
    pltpu.sync_copy(idx_hbm.at[pl.ds(wid * nchunk, nchunk)], idx_v)

    def chunk_dst(j):
        cid = wid * nchunk + j
        ts = cid // (b_tiles * 8)
        tb = (cid // 8) % b_tiles
        r = cid % 8
        return tb * CHUNK, ts * 8 + r

    def start_gather(j, b):
        for g in range(CHUNK // LANES):
            pair_v[b, pl.ds(g * LANES, LANES)] = (
                idx_v[j, pl.ds(g * LANES, LANES)] >> 1
            )
        pltpu.async_copy(table_hbm.at[pair_v.at[b]], rows_v.at[b], gsem.at[b])

    for b in range(PF):
        start_gather(b, b)

    def outer(j0, carry):
        for b in range(NBUF):
            j = j0 + b
            bn = (b + PF) % NBUF
            jn = j + PF

            @pl.when(jn < nchunk)
            def _():
                @pl.when(jn >= NBUF)
                def _():
                    pltpu.make_async_copy(
                        comp_v.at[bn],
                        out_hbm.at[pl.ds(0, CHUNK), 0],
                        ssem.at[bn],
                    ).wait()

                start_gather(jn, bn)

            pltpu.make_async_copy(
                table_hbm.at[pair_v.at[b]],
                rows_v.at[b],
                gsem.at[b],
            ).wait()
            for rg in range(CHUNK // LANES):
                iv = idx_v[j, pl.ds(rg * LANES, LANES)]
                m16 = jnp.where(iv == 0, 0.0, SCALE).astype(jnp.float32)
                o16 = (iv & 1) * NUM_UNITS
                for i in range(LANES):
                    r = rg * LANES + i
                    m = jnp.full((LANES,), m16[i], dtype=jnp.float32)
                    off = o16[i]
                    for c in range(0, NUM_UNITS, LANES):
                        comp_v[b, r, pl.ds(c, LANES)] = (
                            rows_v[b, r, pl.ds(off + c, LANES)] * m
                        )
            b0, s0 = chunk_dst(j)
            pltpu.async_copy(
                comp_v.at[b],
                out_hbm.at[pl.ds(b0, CHUNK), s0],
                ssem.at[b],
            )
        return carry

    lax.fori_loop(0, nchunk // NBUF, lambda t, c: outer(t * NBUF, c), 0)

    for b in range(NBUF):
        pltpu.make_async_copy(
            comp_v.at[b],
            out_hbm.at[pl.ds(0, CHUNK), 0],
            ssem.at[b],
        ).wait()


def kernel(inputs, shared_weights):
    n_b, n_s = inputs.shape
    n_chunks = n_b * n_s // CHUNK
    idx_native = (
        inputs.T.reshape(n_s // 8, 8, n_b // CHUNK, CHUNK)
        .transpose(0, 2, 1, 3)
        .reshape(n_chunks, CHUNK)
    )
    table_pairs = shared_weights.reshape(
        shared_weights.shape[0] // 2, 2 * NUM_UNITS
    )

    mesh = plsc.VectorSubcoreMesh(core_axis_name="c", subcore_axis_name="s")
    run = pl.kernel(
        _sc_body,
        out_type=jax.ShapeDtypeStruct((n_b, n_s, NUM_UNITS), jnp.float32),
        mesh=mesh,
        scratch_types=[
            pltpu.VMEM((n_chunks // NW, CHUNK), jnp.int32),
            pltpu.VMEM((NBUF, CHUNK), jnp.int32),
            pltpu.VMEM((NBUF, CHUNK, 2 * NUM_UNITS), jnp.float32),
            pltpu.VMEM((NBUF, CHUNK, NUM_UNITS), jnp.float32),
            pltpu.SemaphoreType.DMA((NBUF,)),
            pltpu.SemaphoreType.DMA((NBUF,)),
        ],
        compiler_params=pltpu.CompilerParams(use_tc_tiling_on_sc=True),
    )
    return run(idx_native, table_pairs)


# speedup vs baseline: 1.2929x; 1.2929x over previous
import functools

import jax
import jax.numpy as jnp
from jax import lax
from jax.experimental import pallas as pl
from jax.experimental.pallas import tpu as pltpu
from jax.experimental.pallas import tpu_sc as plsc

NUM_UNITS = 64
SCALE = 8.0
NW = 32
CHUNK = 128
LANES = 16
NBUF = 8
PF = 4


def _sc_body(idx_hbm, table_hbm, out_hbm, idx_v, rows_v, gsem, ssem):
    nchunk = idx_v.shape[0] // CHUNK
    wid = lax.axis_index("s") * 2 + lax.axis_index("c")
    pltpu.sync_copy(idx_hbm.at[wid], idx_v)

    def start_gather(j, b):
        pltpu.async_copy(
            table_hbm.at[idx_v.at[pl.ds(j * CHUNK, CHUNK)]],
            rows_v.at[b],
            gsem.at[b],
        )

    for b in range(PF):
        start_gather(b, b)

    def outer(j0, carry):
        for b in range(NBUF):
            j = j0 + b
            bn = (b + PF) % NBUF
            jn = j + PF

            @pl.when(jn < nchunk)
            def _():
                @pl.when(jn >= NBUF)
                def _():
                    pltpu.make_async_copy(
                        rows_v.at[bn], out_hbm.at[wid, 0], ssem.at[bn]
                    ).wait()

                start_gather(jn, bn)

            pltpu.make_async_copy(
                table_hbm.at[idx_v.at[pl.ds(j * CHUNK, CHUNK)]],
                rows_v.at[b],
                gsem.at[b],
            ).wait()
            for rg in range(CHUNK // LANES):
                iv = idx_v[pl.ds(j * CHUNK + rg * LANES, LANES)]
                m16 = jnp.where(iv == 0, 0.0, SCALE).astype(jnp.float32)
                for i in range(LANES):
                    r = rg * LANES + i
                    m = jnp.full((LANES,), m16[i], dtype=jnp.float32)
                    for c in range(0, NUM_UNITS, LANES):
                        rows_v[b, r, pl.ds(c, LANES)] = (
                            rows_v[b, r, pl.ds(c, LANES)] * m
                        )
            pltpu.async_copy(rows_v.at[b], out_hbm.at[wid, j], ssem.at[b])
        return carry

    lax.fori_loop(0, nchunk // NBUF, lambda t, c: outer(t * NBUF, c), 0)

    for b in range(NBUF):
        pltpu.make_async_copy(
            rows_v.at[b], out_hbm.at[wid, 0], ssem.at[b]
        ).wait()


def kernel(inputs, shared_weights):
    n_tok = inputs.shape[0] * inputs.shape[1]
    per_w = n_tok // NW
    nchunk = per_w // CHUNK
    idx3 = inputs.reshape(NW, per_w)

    mesh = plsc.VectorSubcoreMesh(core_axis_name="c", subcore_axis_name="s")
    run = pl.kernel(
        _sc_body,
        out_type=jax.ShapeDtypeStruct((NW, nchunk, CHUNK, NUM_UNITS), jnp.float32),
        mesh=mesh,
        scratch_types=[
            pltpu.VMEM((per_w,), jnp.int32),
            pltpu.VMEM((NBUF, CHUNK, NUM_UNITS), jnp.float32),
            pltpu.SemaphoreType.DMA((NBUF,)),
            pltpu.SemaphoreType.DMA((NBUF,)),
        ],
        compiler_params=pltpu.CompilerParams(use_tc_tiling_on_sc=False),
    )
    out = run(idx3, shared_weights)
    return out.reshape(inputs.shape[0], inputs.shape[1], NUM_UNITS)
